# baseline (speedup 1.0000x reference)
import functools

import jax
import jax.numpy as jnp
from jax.experimental import pallas as pl
from jax.experimental.pallas import tpu as pltpu


def _sigmoid(x):
    return 0.5 * jnp.tanh(0.5 * x) + 0.5


def _lstm_kernel(emb_ref, h0_ref, c0_ref, wih_ref, whh_ref, b_ref,
                 out_ref, hn_ref, cn_ref, gx_ref, *, ts, bt, hidden_size):
    H = hidden_size
    Bt = bt

    @pl.when(pl.program_id(1) == 0)
    def _():
        hn_ref[...] = h0_ref[...]
        cn_ref[...] = c0_ref[...]

    gx_ref[...] = jnp.dot(
        emb_ref[...].reshape(ts * Bt, emb_ref.shape[-1]), wih_ref[...],
        preferred_element_type=jnp.float32) + b_ref[...]

    h = hn_ref[...]
    c = cn_ref[...]
    for i in range(ts):
        gates = gx_ref[i * Bt:(i + 1) * Bt, :] + jnp.dot(
            h.astype(jnp.bfloat16), whh_ref[...],
            preferred_element_type=jnp.float32)
        i_g = _sigmoid(gates[:, 0 * H:1 * H])
        f_g = _sigmoid(gates[:, 1 * H:2 * H])
        g_g = jnp.tanh(gates[:, 2 * H:3 * H])
        o_g = _sigmoid(gates[:, 3 * H:4 * H])
        c = f_g * c + i_g * g_g
        h = o_g * jnp.tanh(c)
        out_ref[i, :, :] = h.astype(out_ref.dtype)

    hn_ref[...] = h
    cn_ref[...] = c


def _lstm_forward(emb, h0, c0, wih, whh, b_gates, *, ts=8, batch_tiles=1):
    S, B, E = emb.shape
    H = h0.shape[-1]
    G = 4 * H
    ts = min(ts, S)
    while S % ts:
        ts //= 2
    while B % batch_tiles or (B // batch_tiles) % 8:
        batch_tiles //= 2
    Bt = B // batch_tiles
    body = functools.partial(_lstm_kernel, ts=ts, bt=Bt, hidden_size=H)
    out, hn, cn = pl.pallas_call(
        body,
        out_shape=[
            jax.ShapeDtypeStruct((S, B, H), jnp.bfloat16),
            jax.ShapeDtypeStruct((B, H), jnp.float32),
            jax.ShapeDtypeStruct((B, H), jnp.float32),
        ],
        grid_spec=pltpu.PrefetchScalarGridSpec(
            num_scalar_prefetch=0,
            grid=(batch_tiles, S // ts),
            in_specs=[
                pl.BlockSpec((ts, Bt, E), lambda i, t: (t, i, 0)),
                pl.BlockSpec((Bt, H), lambda i, t: (i, 0)),
                pl.BlockSpec((Bt, H), lambda i, t: (i, 0)),
                pl.BlockSpec((E, G), lambda i, t: (0, 0)),
                pl.BlockSpec((H, G), lambda i, t: (0, 0)),
                pl.BlockSpec((1, G), lambda i, t: (0, 0)),
            ],
            out_specs=[
                pl.BlockSpec((ts, Bt, H), lambda i, t: (t, i, 0)),
                pl.BlockSpec((Bt, H), lambda i, t: (i, 0)),
                pl.BlockSpec((Bt, H), lambda i, t: (i, 0)),
            ],
            scratch_shapes=[pltpu.VMEM((ts * Bt, G), jnp.float32)],
        ),
        compiler_params=pltpu.CompilerParams(
            dimension_semantics=("parallel", "arbitrary")),
    )(emb, h0, c0, wih, whh, b_gates)
    return out, hn, cn


def _head_kernel(x_ref, w_ref, b_ref, o_ref):
    o_ref[...] = jnp.dot(x_ref[...], w_ref[...],
                         preferred_element_type=jnp.float32) + b_ref[...]


def _head(x, w, b, *, tm=4096, tn=1024):
    N, K = x.shape
    V = w.shape[1]
    tm, tn = min(tm, N), min(tn, V)
    while N % tm:
        tm //= 2
    while V % tn:
        tn //= 2
    return pl.pallas_call(
        _head_kernel,
        out_shape=jax.ShapeDtypeStruct((N, V), jnp.float32),
        grid_spec=pltpu.PrefetchScalarGridSpec(
            num_scalar_prefetch=0,
            grid=(N // tm, V // tn),
            in_specs=[
                pl.BlockSpec((tm, K), lambda i, j: (i, 0)),
                pl.BlockSpec((K, tn), lambda i, j: (0, j)),
                pl.BlockSpec((1, tn), lambda i, j: (0, j)),
            ],
            out_specs=pl.BlockSpec((tm, tn), lambda i, j: (i, j)),
        ),
        compiler_params=pltpu.CompilerParams(
            dimension_semantics=("parallel", "parallel")),
    )(x, w, b)


def _gather_kernel(ids_ref, tab_ref, out_ref, *, rows, unroll):
    base = pl.program_id(0) * rows

    def body(m, carry):
        for u in range(unroll):
            k = m * unroll + u
            idx = ids_ref[base + k]
            out_ref[pl.ds(k, 1), :] = tab_ref[pl.ds(idx, 1), :]
        return carry

    jax.lax.fori_loop(0, rows // unroll, body, 0)


def _embed_gather(tab_i32, ids, *, rows=512, unroll=8):
    N = ids.shape[0]
    W = tab_i32.shape[1]
    rows = min(rows, N)
    grid = (N // rows,)
    body = functools.partial(_gather_kernel, rows=rows, unroll=unroll)
    return pl.pallas_call(
        body,
        out_shape=jax.ShapeDtypeStruct((N, W), jnp.int32),
        grid_spec=pltpu.PrefetchScalarGridSpec(
            num_scalar_prefetch=1,
            grid=grid,
            in_specs=[
                pl.BlockSpec(tab_i32.shape, lambda g, ids: (0, 0)),
            ],
            out_specs=pl.BlockSpec((rows, W), lambda g, ids: (g, 0)),
        ),
        compiler_params=pltpu.CompilerParams(
            dimension_semantics=("arbitrary",)),
    )(ids, tab_i32)


def kernel(embed_w, wih, whh, b_gates, lin_w_t, lin_b, x, h0, c0):
    S, B = x.shape
    H = h0.shape[-1]
    V, E = embed_w.shape
    tab_i32 = jax.lax.bitcast_convert_type(
        embed_w.reshape(V, E // 2, 2), jnp.int32)
    emb_i32 = _embed_gather(tab_i32, x.reshape(S * B))
    emb = jax.lax.bitcast_convert_type(emb_i32, jnp.bfloat16).reshape(S, B, E)
    out, hn, cn = _lstm_forward(emb, h0[0], c0[0], wih, whh, b_gates)
    logits = _head(out.reshape(S * B, H), lin_w_t, lin_b)
    return logits, (hn[None, :, :], cn[None, :, :])


# speedup vs baseline: 2.2971x; 2.2971x over previous
import functools

import jax
import jax.numpy as jnp
from jax.experimental import pallas as pl
from jax.experimental.pallas import tpu as pltpu


def _sigmoid(x):
    return 0.5 * jnp.tanh(0.5 * x) + 0.5


def _lstm_kernel(emb_ref, h0_ref, c0_ref, wih_ref, whh_ref, b_ref,
                 out_ref, hn_ref, cn_ref, gx_ref, *, ts, bt, hidden_size):
    H = hidden_size
    Bt = bt

    @pl.when(pl.program_id(1) == 0)
    def _():
        hn_ref[...] = h0_ref[...]
        cn_ref[...] = c0_ref[...]

    gx_ref[...] = jnp.dot(
        emb_ref[...].reshape(ts * Bt, emb_ref.shape[-1]), wih_ref[...],
        preferred_element_type=jnp.float32) + b_ref[...]

    h = hn_ref[...]
    c = cn_ref[...]
    for i in range(ts):
        gates = gx_ref[i * Bt:(i + 1) * Bt, :] + jnp.dot(
            h.astype(jnp.bfloat16), whh_ref[...],
            preferred_element_type=jnp.float32)
        i_g = _sigmoid(gates[:, 0 * H:1 * H])
        f_g = _sigmoid(gates[:, 1 * H:2 * H])
        g_g = jnp.tanh(gates[:, 2 * H:3 * H])
        o_g = _sigmoid(gates[:, 3 * H:4 * H])
        c = f_g * c + i_g * g_g
        h = o_g * jnp.tanh(c)
        out_ref[i, :, :] = h.astype(out_ref.dtype)

    hn_ref[...] = h
    cn_ref[...] = c


def _lstm_forward(emb, h0, c0, wih, whh, b_gates, *, ts=8, batch_tiles=1):
    S, B, E = emb.shape
    H = h0.shape[-1]
    G = 4 * H
    ts = min(ts, S)
    while S % ts:
        ts //= 2
    while B % batch_tiles or (B // batch_tiles) % 8:
        batch_tiles //= 2
    Bt = B // batch_tiles
    body = functools.partial(_lstm_kernel, ts=ts, bt=Bt, hidden_size=H)
    out, hn, cn = pl.pallas_call(
        body,
        out_shape=[
            jax.ShapeDtypeStruct((S, B, H), jnp.bfloat16),
            jax.ShapeDtypeStruct((B, H), jnp.float32),
            jax.ShapeDtypeStruct((B, H), jnp.float32),
        ],
        grid_spec=pltpu.PrefetchScalarGridSpec(
            num_scalar_prefetch=0,
            grid=(batch_tiles, S // ts),
            in_specs=[
                pl.BlockSpec((ts, Bt, E), lambda i, t: (t, i, 0)),
                pl.BlockSpec((Bt, H), lambda i, t: (i, 0)),
                pl.BlockSpec((Bt, H), lambda i, t: (i, 0)),
                pl.BlockSpec((E, G), lambda i, t: (0, 0)),
                pl.BlockSpec((H, G), lambda i, t: (0, 0)),
                pl.BlockSpec((1, G), lambda i, t: (0, 0)),
            ],
            out_specs=[
                pl.BlockSpec((ts, Bt, H), lambda i, t: (t, i, 0)),
                pl.BlockSpec((Bt, H), lambda i, t: (i, 0)),
                pl.BlockSpec((Bt, H), lambda i, t: (i, 0)),
            ],
            scratch_shapes=[pltpu.VMEM((ts * Bt, G), jnp.float32)],
        ),
        compiler_params=pltpu.CompilerParams(
            dimension_semantics=("parallel", "arbitrary")),
    )(emb, h0, c0, wih, whh, b_gates)
    return out, hn, cn


def _head_kernel(x_ref, w_ref, b_ref, o_ref):
    o_ref[...] = jnp.dot(x_ref[...], w_ref[...],
                         preferred_element_type=jnp.float32) + b_ref[...]


def _head(x, w, b, *, tm=4096, tn=1024):
    N, K = x.shape
    V = w.shape[1]
    tm, tn = min(tm, N), min(tn, V)
    while N % tm:
        tm //= 2
    while V % tn:
        tn //= 2
    return pl.pallas_call(
        _head_kernel,
        out_shape=jax.ShapeDtypeStruct((N, V), jnp.float32),
        grid_spec=pltpu.PrefetchScalarGridSpec(
            num_scalar_prefetch=0,
            grid=(N // tm, V // tn),
            in_specs=[
                pl.BlockSpec((tm, K), lambda i, j: (i, 0)),
                pl.BlockSpec((K, tn), lambda i, j: (0, j)),
                pl.BlockSpec((1, tn), lambda i, j: (0, j)),
            ],
            out_specs=pl.BlockSpec((tm, tn), lambda i, j: (i, j)),
        ),
        compiler_params=pltpu.CompilerParams(
            dimension_semantics=("parallel", "parallel")),
    )(x, w, b)


def _gather_kernel(ids_ref, tab_ref, out_ref, *, rows, unroll):
    del unroll
    base = pl.program_id(0) * rows
    sub = jax.lax.broadcasted_iota(jnp.int32, (8, 1), 0)

    def body(m, carry):
        out_slab = jnp.zeros((8, out_ref.shape[1]), jnp.float32)
        for u in range(8):
            idx = ids_ref[base + m * 8 + u]
            slab = tab_ref[pl.ds(pl.multiple_of((idx >> 3) << 3, 8), 8), :]
            row = jnp.sum(jnp.where(sub == (idx & 7),
                                    slab.astype(jnp.float32), 0.0),
                          axis=0, keepdims=True)
            out_slab = out_slab + jnp.where(sub == u, row, 0.0)
        out_ref[pl.ds(pl.multiple_of(m * 8, 8), 8), :] = \
            out_slab.astype(out_ref.dtype)
        return carry

    jax.lax.fori_loop(0, rows // 8, body, 0)


def _embed_gather(tab, ids, *, rows=512, unroll=8):
    N = ids.shape[0]
    W = tab.shape[1]
    rows = min(rows, N)
    grid = (N // rows,)
    body = functools.partial(_gather_kernel, rows=rows, unroll=unroll)
    return pl.pallas_call(
        body,
        out_shape=jax.ShapeDtypeStruct((N, W), tab.dtype),
        grid_spec=pltpu.PrefetchScalarGridSpec(
            num_scalar_prefetch=1,
            grid=grid,
            in_specs=[
                pl.BlockSpec(tab.shape, lambda g, ids: (0, 0)),
            ],
            out_specs=pl.BlockSpec((rows, W), lambda g, ids: (g, 0)),
        ),
        compiler_params=pltpu.CompilerParams(
            dimension_semantics=("arbitrary",)),
    )(ids, tab)


def kernel(embed_w, wih, whh, b_gates, lin_w_t, lin_b, x, h0, c0):
    S, B = x.shape
    H = h0.shape[-1]
    V, E = embed_w.shape
    emb = _embed_gather(embed_w, x.reshape(S * B)).reshape(S, B, E)
    out, hn, cn = _lstm_forward(emb, h0[0], c0[0], wih, whh, b_gates)
    logits = _head(out.reshape(S * B, H), lin_w_t, lin_b)
    return logits, (hn[None, :, :], cn[None, :, :])


# speedup vs baseline: 2.3108x; 1.0060x over previous
import functools

import jax
import jax.numpy as jnp
from jax.experimental import pallas as pl
from jax.experimental.pallas import tpu as pltpu


def _sigmoid(x):
    return 0.5 * jnp.tanh(0.5 * x) + 0.5


def _lstm_kernel(ids_ref, tab_ref, h0_ref, c0_ref, wih_ref, whh_ref, b_ref,
                 out_ref, hn_ref, cn_ref, slab_ref, gx_ref, *,
                 ts, batch, hidden_size):
    H = hidden_size
    B = batch
    R = ts * B
    E = tab_ref.shape[1]
    t = pl.program_id(0)
    nt = pl.num_programs(0)
    sub8 = jax.lax.broadcasted_iota(jnp.int32, (8, 1), 0)
    sub16 = jax.lax.broadcasted_iota(jnp.int32, (16, 1), 0)

    def gather_block(tb, slot):
        base = tb * R

        def body(m, carry):
            out_slab = jnp.zeros((16, E), jnp.float32)
            for u in range(16):
                idx = ids_ref[base + m * 16 + u]
                chunk = tab_ref[pl.ds(pl.multiple_of((idx >> 3) << 3, 8), 8), :]
                row = jnp.sum(jnp.where(sub8 == (idx & 7),
                                        chunk.astype(jnp.float32), 0.0),
                              axis=0, keepdims=True)
                out_slab = out_slab + jnp.where(sub16 == u, row, 0.0)
            slab_ref[slot, pl.ds(pl.multiple_of(m * 16, 16), 16), :] = \
                out_slab.astype(slab_ref.dtype)
            return carry

        jax.lax.fori_loop(0, R // 16, body, 0)

    @pl.when(t == 0)
    def _():
        hn_ref[...] = h0_ref[...]
        cn_ref[...] = c0_ref[...]
        gather_block(0, 0)

    nxt = jnp.minimum(t + 1, nt - 1)
    gather_block(nxt, nxt % 2)

    gx_ref[...] = jnp.dot(slab_ref[t % 2], wih_ref[...],
                          preferred_element_type=jnp.float32) + b_ref[...]

    h = hn_ref[...]
    c = cn_ref[...]
    for i in range(ts):
        gates = gx_ref[i * B:(i + 1) * B, :] + jnp.dot(
            h.astype(jnp.bfloat16), whh_ref[...],
            preferred_element_type=jnp.float32)
        i_g = _sigmoid(gates[:, 0 * H:1 * H])
        f_g = _sigmoid(gates[:, 1 * H:2 * H])
        g_g = jnp.tanh(gates[:, 2 * H:3 * H])
        o_g = _sigmoid(gates[:, 3 * H:4 * H])
        c = f_g * c + i_g * g_g
        h = o_g * jnp.tanh(c)
        out_ref[i, :, :] = h.astype(out_ref.dtype)

    hn_ref[...] = h
    cn_ref[...] = c


def _lstm_forward(tab, ids, h0, c0, wih, whh, b_gates, *, seq, batch, ts=8):
    S, B = seq, batch
    E = tab.shape[1]
    H = h0.shape[-1]
    G = 4 * H
    ts = min(ts, S)
    while S % ts:
        ts //= 2
    body = functools.partial(_lstm_kernel, ts=ts, batch=B, hidden_size=H)
    out, hn, cn = pl.pallas_call(
        body,
        out_shape=[
            jax.ShapeDtypeStruct((S, B, H), jnp.bfloat16),
            jax.ShapeDtypeStruct((B, H), jnp.float32),
            jax.ShapeDtypeStruct((B, H), jnp.float32),
        ],
        grid_spec=pltpu.PrefetchScalarGridSpec(
            num_scalar_prefetch=1,
            grid=(S // ts,),
            in_specs=[
                pl.BlockSpec(tab.shape, lambda t, ids: (0, 0)),
                pl.BlockSpec((B, H), lambda t, ids: (0, 0)),
                pl.BlockSpec((B, H), lambda t, ids: (0, 0)),
                pl.BlockSpec((E, G), lambda t, ids: (0, 0)),
                pl.BlockSpec((H, G), lambda t, ids: (0, 0)),
                pl.BlockSpec((1, G), lambda t, ids: (0, 0)),
            ],
            out_specs=[
                pl.BlockSpec((ts, B, H), lambda t, ids: (t, 0, 0)),
                pl.BlockSpec((B, H), lambda t, ids: (0, 0)),
                pl.BlockSpec((B, H), lambda t, ids: (0, 0)),
            ],
            scratch_shapes=[
                pltpu.VMEM((2, ts * B, E), jnp.bfloat16),
                pltpu.VMEM((ts * B, G), jnp.float32),
            ],
        ),
        compiler_params=pltpu.CompilerParams(
            dimension_semantics=("arbitrary",)),
    )(ids, tab, h0, c0, wih, whh, b_gates)
    return out, hn, cn


def _head_kernel(x_ref, w_ref, b_ref, o_ref):
    o_ref[...] = jnp.dot(x_ref[...], w_ref[...],
                         preferred_element_type=jnp.float32) + b_ref[...]


def _head(x, w, b, *, tm=4096, tn=1024):
    N, K = x.shape
    V = w.shape[1]
    tm, tn = min(tm, N), min(tn, V)
    while N % tm:
        tm //= 2
    while V % tn:
        tn //= 2
    return pl.pallas_call(
        _head_kernel,
        out_shape=jax.ShapeDtypeStruct((N, V), jnp.float32),
        grid_spec=pltpu.PrefetchScalarGridSpec(
            num_scalar_prefetch=0,
            grid=(V // tn, N // tm),
            in_specs=[
                pl.BlockSpec((tm, K), lambda j, i: (i, 0)),
                pl.BlockSpec((K, tn), lambda j, i: (0, j)),
                pl.BlockSpec((1, tn), lambda j, i: (0, j)),
            ],
            out_specs=pl.BlockSpec((tm, tn), lambda j, i: (i, j)),
        ),
        compiler_params=pltpu.CompilerParams(
            dimension_semantics=("parallel", "parallel")),
    )(x, w, b)


def kernel(embed_w, wih, whh, b_gates, lin_w_t, lin_b, x, h0, c0):
    S, B = x.shape
    H = h0.shape[-1]
    out, hn, cn = _lstm_forward(embed_w, x.reshape(S * B), h0[0], c0[0],
                                wih, whh, b_gates, seq=S, batch=B)
    logits = _head(out.reshape(S * B, H), lin_w_t, lin_b)
    return logits, (hn[None, :, :], cn[None, :, :])


# speedup vs baseline: 2.3461x; 1.0153x over previous
import functools

import jax
import jax.numpy as jnp
from jax.experimental import pallas as pl
from jax.experimental.pallas import tpu as pltpu


def _sigmoid(x):
    return 0.5 * jnp.tanh(0.5 * x) + 0.5


def _lstm_kernel(ids_ref, tab_ref, h0_ref, c0_ref, wih_ref, whh_ref, b_ref,
                 out_ref, hn_ref, cn_ref, slab_ref, gx_ref, *,
                 ts, batch, hidden_size):
    H = hidden_size
    B = batch
    R = ts * B
    E = tab_ref.shape[1]
    t = pl.program_id(0)
    nt = pl.num_programs(0)
    sub8 = jax.lax.broadcasted_iota(jnp.int32, (8, 1), 0)
    sub16 = jax.lax.broadcasted_iota(jnp.int32, (16, 1), 0)

    def gather_block(tb, slot):
        base = tb * R
        for m in range(R // 16):
            out_slab = jnp.zeros((16, E), jnp.float32)
            for u in range(16):
                idx = ids_ref[base + m * 16 + u]
                chunk = tab_ref[pl.ds(pl.multiple_of((idx >> 3) << 3, 8), 8), :]
                row = jnp.sum(jnp.where(sub8 == (idx & 7),
                                        chunk.astype(jnp.float32), 0.0),
                              axis=0, keepdims=True)
                out_slab = out_slab + jnp.where(sub16 == u, row, 0.0)
            slab_ref[slot, m * 16:(m + 1) * 16, :] = \
                out_slab.astype(slab_ref.dtype)

    @pl.when(t == 0)
    def _():
        hn_ref[...] = h0_ref[...]
        cn_ref[...] = c0_ref[...]
        gather_block(0, 0)

    nxt = jnp.minimum(t + 1, nt - 1)
    gather_block(nxt, nxt % 2)

    gx_ref[...] = jnp.dot(slab_ref[t % 2], wih_ref[...],
                          preferred_element_type=jnp.float32) + b_ref[...]

    h = hn_ref[...]
    c = cn_ref[...]
    for i in range(ts):
        gates = gx_ref[i * B:(i + 1) * B, :] + jnp.dot(
            h.astype(jnp.bfloat16), whh_ref[...],
            preferred_element_type=jnp.float32)
        i_g = _sigmoid(gates[:, 0 * H:1 * H])
        f_g = _sigmoid(gates[:, 1 * H:2 * H])
        g_g = jnp.tanh(gates[:, 2 * H:3 * H])
        o_g = _sigmoid(gates[:, 3 * H:4 * H])
        c = f_g * c + i_g * g_g
        h = o_g * jnp.tanh(c)
        out_ref[i, :, :] = h.astype(out_ref.dtype)

    hn_ref[...] = h
    cn_ref[...] = c


def _lstm_forward(tab, ids, h0, c0, wih, whh, b_gates, *, seq, batch, ts=8):
    S, B = seq, batch
    E = tab.shape[1]
    H = h0.shape[-1]
    G = 4 * H
    ts = min(ts, S)
    while S % ts:
        ts //= 2
    body = functools.partial(_lstm_kernel, ts=ts, batch=B, hidden_size=H)
    out, hn, cn = pl.pallas_call(
        body,
        out_shape=[
            jax.ShapeDtypeStruct((S, B, H), jnp.bfloat16),
            jax.ShapeDtypeStruct((B, H), jnp.float32),
            jax.ShapeDtypeStruct((B, H), jnp.float32),
        ],
        grid_spec=pltpu.PrefetchScalarGridSpec(
            num_scalar_prefetch=1,
            grid=(S // ts,),
            in_specs=[
                pl.BlockSpec(tab.shape, lambda t, ids: (0, 0)),
                pl.BlockSpec((B, H), lambda t, ids: (0, 0)),
                pl.BlockSpec((B, H), lambda t, ids: (0, 0)),
                pl.BlockSpec((E, G), lambda t, ids: (0, 0)),
                pl.BlockSpec((H, G), lambda t, ids: (0, 0)),
                pl.BlockSpec((1, G), lambda t, ids: (0, 0)),
            ],
            out_specs=[
                pl.BlockSpec((ts, B, H), lambda t, ids: (t, 0, 0)),
                pl.BlockSpec((B, H), lambda t, ids: (0, 0)),
                pl.BlockSpec((B, H), lambda t, ids: (0, 0)),
            ],
            scratch_shapes=[
                pltpu.VMEM((2, ts * B, E), jnp.bfloat16),
                pltpu.VMEM((ts * B, G), jnp.float32),
            ],
        ),
        compiler_params=pltpu.CompilerParams(
            dimension_semantics=("arbitrary",)),
    )(ids, tab, h0, c0, wih, whh, b_gates)
    return out, hn, cn


def _head_kernel(x_ref, w_ref, b_ref, o_ref):
    o_ref[...] = jnp.dot(x_ref[...], w_ref[...],
                         preferred_element_type=jnp.float32) + b_ref[...]


def _head(x, w, b, *, tm=4096, tn=1024):
    N, K = x.shape
    V = w.shape[1]
    tm, tn = min(tm, N), min(tn, V)
    while N % tm:
        tm //= 2
    while V % tn:
        tn //= 2
    return pl.pallas_call(
        _head_kernel,
        out_shape=jax.ShapeDtypeStruct((N, V), jnp.float32),
        grid_spec=pltpu.PrefetchScalarGridSpec(
            num_scalar_prefetch=0,
            grid=(V // tn, N // tm),
            in_specs=[
                pl.BlockSpec((tm, K), lambda j, i: (i, 0)),
                pl.BlockSpec((K, tn), lambda j, i: (0, j)),
                pl.BlockSpec((1, tn), lambda j, i: (0, j)),
            ],
            out_specs=pl.BlockSpec((tm, tn), lambda j, i: (i, j)),
        ),
        compiler_params=pltpu.CompilerParams(
            dimension_semantics=("parallel", "parallel")),
    )(x, w, b)


def kernel(embed_w, wih, whh, b_gates, lin_w_t, lin_b, x, h0, c0):
    S, B = x.shape
    H = h0.shape[-1]
    out, hn, cn = _lstm_forward(embed_w, x.reshape(S * B), h0[0], c0[0],
                                wih, whh, b_gates, seq=S, batch=B)
    logits = _head(out.reshape(S * B, H), lin_w_t, lin_b)
    return logits, (hn[None, :, :], cn[None, :, :])


# speedup vs baseline: 2.4626x; 1.0497x over previous
import functools

import jax
import jax.numpy as jnp
from jax.experimental import pallas as pl
from jax.experimental.pallas import tpu as pltpu


def _sigmoid(x):
    return 0.5 * jnp.tanh(0.5 * x) + 0.5


def _lstm_kernel(ids_ref, tab_ref, h0_ref, c0_ref, wih_ref, whh_ref, b_ref,
                 out_ref, hn_ref, cn_ref, slab_ref, gx_ref, *,
                 ts, batch, hidden_size):
    H = hidden_size
    B = batch
    R = ts * B
    E = tab_ref.shape[1]
    t = pl.program_id(0)
    nt = pl.num_programs(0)
    tabi_ref = tab_ref.bitcast(jnp.int32)

    def gather_block(tb, slot):
        base = tb * R
        for m in range(R // 16):
            words = []
            for p in range(8):
                halves = []
                for q in range(2):
                    idx = ids_ref[base + m * 16 + 2 * p + q]
                    rowv = tabi_ref[pl.ds(idx >> 1, 1), :]
                    halves.append((rowv >> ((idx & 1) * 16)) & 0xFFFF)
                words.append(halves[0] | (halves[1] << 16))
            grp = jnp.concatenate(words, axis=0)
            slab_ref[slot, pl.ds(pl.multiple_of(m * 8, 8), 8), :] = grp

    @pl.when(t == 0)
    def _():
        hn_ref[...] = h0_ref[...]
        cn_ref[...] = c0_ref[...]
        gather_block(0, 0)

    nxt = jnp.minimum(t + 1, nt - 1)
    gather_block(nxt, nxt % 2)

    emb_blk = pltpu.bitcast(slab_ref[t % 2], jnp.bfloat16)
    gx_ref[...] = jnp.dot(emb_blk, wih_ref[...],
                          preferred_element_type=jnp.float32) + b_ref[...]

    h = hn_ref[...]
    c = cn_ref[...]
    for i in range(ts):
        gates = gx_ref[i * B:(i + 1) * B, :] + jnp.dot(
            h.astype(jnp.bfloat16), whh_ref[...],
            preferred_element_type=jnp.float32)
        i_g = _sigmoid(gates[:, 0 * H:1 * H])
        f_g = _sigmoid(gates[:, 1 * H:2 * H])
        g_g = jnp.tanh(gates[:, 2 * H:3 * H])
        o_g = _sigmoid(gates[:, 3 * H:4 * H])
        c = f_g * c + i_g * g_g
        h = o_g * jnp.tanh(c)
        out_ref[i, :, :] = h.astype(out_ref.dtype)

    hn_ref[...] = h
    cn_ref[...] = c


def _lstm_forward(tab, ids, h0, c0, wih, whh, b_gates, *, seq, batch, ts=8):
    S, B = seq, batch
    E = tab.shape[1]
    H = h0.shape[-1]
    G = 4 * H
    ts = min(ts, S)
    while S % ts:
        ts //= 2
    body = functools.partial(_lstm_kernel, ts=ts, batch=B, hidden_size=H)
    out, hn, cn = pl.pallas_call(
        body,
        out_shape=[
            jax.ShapeDtypeStruct((S, B, H), jnp.bfloat16),
            jax.ShapeDtypeStruct((B, H), jnp.float32),
            jax.ShapeDtypeStruct((B, H), jnp.float32),
        ],
        grid_spec=pltpu.PrefetchScalarGridSpec(
            num_scalar_prefetch=1,
            grid=(S // ts,),
            in_specs=[
                pl.BlockSpec(tab.shape, lambda t, ids: (0, 0)),
                pl.BlockSpec((B, H), lambda t, ids: (0, 0)),
                pl.BlockSpec((B, H), lambda t, ids: (0, 0)),
                pl.BlockSpec((E, G), lambda t, ids: (0, 0)),
                pl.BlockSpec((H, G), lambda t, ids: (0, 0)),
                pl.BlockSpec((1, G), lambda t, ids: (0, 0)),
            ],
            out_specs=[
                pl.BlockSpec((ts, B, H), lambda t, ids: (t, 0, 0)),
                pl.BlockSpec((B, H), lambda t, ids: (0, 0)),
                pl.BlockSpec((B, H), lambda t, ids: (0, 0)),
            ],
            scratch_shapes=[
                pltpu.VMEM((2, ts * B // 2, E), jnp.int32),
                pltpu.VMEM((ts * B, G), jnp.float32),
            ],
        ),
        compiler_params=pltpu.CompilerParams(
            dimension_semantics=("arbitrary",)),
    )(ids, tab, h0, c0, wih, whh, b_gates)
    return out, hn, cn


def _head_kernel(x_ref, w_ref, b_ref, o_ref):
    o_ref[...] = jnp.dot(x_ref[...], w_ref[...],
                         preferred_element_type=jnp.float32) + b_ref[...]


def _head(x, w, b, *, tm=4096, tn=1024):
    N, K = x.shape
    V = w.shape[1]
    tm, tn = min(tm, N), min(tn, V)
    while N % tm:
        tm //= 2
    while V % tn:
        tn //= 2
    return pl.pallas_call(
        _head_kernel,
        out_shape=jax.ShapeDtypeStruct((N, V), jnp.float32),
        grid_spec=pltpu.PrefetchScalarGridSpec(
            num_scalar_prefetch=0,
            grid=(V // tn, N // tm),
            in_specs=[
                pl.BlockSpec((tm, K), lambda j, i: (i, 0)),
                pl.BlockSpec((K, tn), lambda j, i: (0, j)),
                pl.BlockSpec((1, tn), lambda j, i: (0, j)),
            ],
            out_specs=pl.BlockSpec((tm, tn), lambda j, i: (i, j)),
        ),
        compiler_params=pltpu.CompilerParams(
            dimension_semantics=("parallel", "parallel")),
    )(x, w, b)


def kernel(embed_w, wih, whh, b_gates, lin_w_t, lin_b, x, h0, c0):
    S, B = x.shape
    H = h0.shape[-1]
    out, hn, cn = _lstm_forward(embed_w, x.reshape(S * B), h0[0], c0[0],
                                wih, whh, b_gates, seq=S, batch=B)
    logits = _head(out.reshape(S * B, H), lin_w_t, lin_b)
    return logits, (hn[None, :, :], cn[None, :, :])
